# baseline (speedup 1.0000x reference)
import jax
import jax.numpy as jnp
from jax.experimental import pallas as pl
from jax.experimental.pallas import tpu as pltpu

_N_CHUNKS = 16
_ROWS = 12288
_CHUNK_ROWS = _ROWS // _N_CHUNKS


def _copy_kernel(in_ref, out_ref, buf, in_sems, out_sems):
    for i in range(_N_CHUNKS):
        pltpu.make_async_copy(
            in_ref.at[pl.ds(i * _CHUNK_ROWS, _CHUNK_ROWS)], buf.at[i], in_sems.at[i]
        ).start()
    for i in range(_N_CHUNKS):
        pltpu.make_async_copy(
            in_ref.at[pl.ds(i * _CHUNK_ROWS, _CHUNK_ROWS)], buf.at[i], in_sems.at[i]
        ).wait()
        pltpu.make_async_copy(
            buf.at[i], out_ref.at[pl.ds(i * _CHUNK_ROWS, _CHUNK_ROWS)], out_sems.at[i]
        ).start()
    for i in range(_N_CHUNKS):
        pltpu.make_async_copy(
            buf.at[i], out_ref.at[pl.ds(i * _CHUNK_ROWS, _CHUNK_ROWS)], out_sems.at[i]
        ).wait()


def kernel(images):
    flat = images.reshape(_ROWS, 512)
    out = pl.pallas_call(
        _copy_kernel,
        out_shape=jax.ShapeDtypeStruct(flat.shape, flat.dtype),
        in_specs=[pl.BlockSpec(memory_space=pl.ANY)],
        out_specs=pl.BlockSpec(memory_space=pl.ANY),
        scratch_shapes=[
            pltpu.VMEM((_N_CHUNKS, _CHUNK_ROWS, 512), jnp.float32),
            pltpu.SemaphoreType.DMA((_N_CHUNKS,)),
            pltpu.SemaphoreType.DMA((_N_CHUNKS,)),
        ],
    )(flat)
    return out.reshape(images.shape)


# speedup vs baseline: 1.0065x; 1.0065x over previous
import jax
import jax.numpy as jnp
from jax.experimental import pallas as pl
from jax.experimental.pallas import tpu as pltpu

_N_CHUNKS = 8
_ROWS = 12288
_CHUNK_ROWS = _ROWS // _N_CHUNKS


def _copy_kernel(in_ref, out_ref, buf, in_sems, out_sems):
    for i in range(_N_CHUNKS):
        pltpu.make_async_copy(
            in_ref.at[pl.ds(i * _CHUNK_ROWS, _CHUNK_ROWS)], buf.at[i], in_sems.at[i]
        ).start()
    for i in range(_N_CHUNKS):
        pltpu.make_async_copy(
            in_ref.at[pl.ds(i * _CHUNK_ROWS, _CHUNK_ROWS)], buf.at[i], in_sems.at[i]
        ).wait()
        pltpu.make_async_copy(
            buf.at[i], out_ref.at[pl.ds(i * _CHUNK_ROWS, _CHUNK_ROWS)], out_sems.at[i]
        ).start()
    for i in range(_N_CHUNKS):
        pltpu.make_async_copy(
            buf.at[i], out_ref.at[pl.ds(i * _CHUNK_ROWS, _CHUNK_ROWS)], out_sems.at[i]
        ).wait()


def kernel(images):
    flat = images.reshape(_ROWS, 512)
    out = pl.pallas_call(
        _copy_kernel,
        out_shape=jax.ShapeDtypeStruct(flat.shape, flat.dtype),
        in_specs=[pl.BlockSpec(memory_space=pl.ANY)],
        out_specs=pl.BlockSpec(memory_space=pl.ANY),
        scratch_shapes=[
            pltpu.VMEM((_N_CHUNKS, _CHUNK_ROWS, 512), jnp.float32),
            pltpu.SemaphoreType.DMA((_N_CHUNKS,)),
            pltpu.SemaphoreType.DMA((_N_CHUNKS,)),
        ],
    )(flat)
    return out.reshape(images.shape)
